# Initial kernel scaffold; baseline (speedup 1.0000x reference)
import jax
import jax.numpy as jnp
from jax.experimental import pallas as pl


def kernel(input, indexes, codes):
    raise NotImplementedError("write your pallas kernel here")


# speedup vs baseline: 296.8976x; 296.8976x over previous
import functools

import jax
import jax.numpy as jnp
from jax import lax
from jax.experimental import pallas as pl
from jax.experimental.pallas import tpu as pltpu
from jax.experimental.pallas import tpu_sc as plsc


def _build_kernel(V, qdim, centroids, chunk, per_w, ch):
    dim = qdim * chunk
    n_chunks = per_w // ch
    mesh = plsc.VectorSubcoreMesh(core_axis_name="c", subcore_axis_name="s")

    @functools.partial(
        pl.kernel,
        mesh=mesh,
        compiler_params=pltpu.CompilerParams(
            needs_layout_passes=False, use_tc_tiling_on_sc=False),
        out_type=jax.ShapeDtypeStruct((V * dim,), jnp.float32),
        scratch_types=[
            pltpu.VMEM((ch,), jnp.int32),
            pltpu.VMEM((ch, qdim), jnp.int32),
            pltpu.VMEM((qdim * centroids * chunk,), jnp.float32),
            pltpu.VMEM((ch * dim,), jnp.float32),
            pltpu.SemaphoreType.DMA,
        ],
    )
    def pq_decode(ids_hbm, indexes_hbm, codes_hbm, out_hbm,
                  ids_v, idx_v, codes_v, out_v, sem):
        nc = 2
        wid = lax.axis_index("s") * nc + lax.axis_index("c")
        base = wid * per_w

        pltpu.sync_copy(codes_hbm, codes_v)

        q_iota = lax.iota(jnp.int32, 16)
        q_base = q_iota * (centroids * chunk)
        o_iota = q_iota * chunk

        def chunk_body(i, carry):
            g0 = base + i * ch
            pltpu.sync_copy(ids_hbm.at[pl.ds(g0, ch)], ids_v)
            pltpu.async_copy(indexes_hbm.at[ids_v], idx_v, sem).wait()

            def id_body(v, c2):
                iv = idx_v[v, :]
                addr0 = q_base + iv * chunk
                obase = o_iota + v * dim
                for c in range(chunk):
                    vals = plsc.load_gather(codes_v, [addr0 + c])
                    plsc.store_scatter(out_v, [obase + c], vals)
                return c2

            lax.fori_loop(0, ch, id_body, 0)
            pltpu.sync_copy(out_v, out_hbm.at[pl.ds(g0 * dim, ch * dim)])
            return carry

        lax.fori_loop(0, n_chunks, chunk_body, 0)

    return pq_decode


def kernel(input, indexes, codes):
    shape = input.shape
    qdim, centroids, chunk = codes.shape
    dim = qdim * chunk
    flat = input.reshape(-1)
    V = flat.shape[0]
    per_w = V // 32
    ch = 128
    fn = _build_kernel(V, qdim, centroids, chunk, per_w, ch)
    out = fn(flat, indexes, codes.reshape(-1))
    return out.reshape(shape + (dim,))


# speedup vs baseline: 336.9212x; 1.1348x over previous
import functools

import jax
import jax.numpy as jnp
from jax import lax
from jax.experimental import pallas as pl
from jax.experimental.pallas import tpu as pltpu
from jax.experimental.pallas import tpu_sc as plsc


def _build_kernel(V, qdim, centroids, chunk, per_w, ch):
    dim = qdim * chunk
    n_chunks = per_w // ch
    assert n_chunks % 2 == 0
    mesh = plsc.VectorSubcoreMesh(core_axis_name="c", subcore_axis_name="s")

    @functools.partial(
        pl.kernel,
        mesh=mesh,
        compiler_params=pltpu.CompilerParams(
            needs_layout_passes=False, use_tc_tiling_on_sc=False),
        out_type=jax.ShapeDtypeStruct((V * dim,), jnp.float32),
        scratch_types=[
            pltpu.VMEM((per_w,), jnp.int32),
            pltpu.VMEM((ch, qdim), jnp.int32),
            pltpu.VMEM((ch, qdim), jnp.int32),
            pltpu.VMEM((qdim * centroids * chunk,), jnp.float32),
            pltpu.VMEM((ch * dim,), jnp.float32),
            pltpu.VMEM((ch * dim,), jnp.float32),
            pltpu.SemaphoreType.DMA,
            pltpu.SemaphoreType.DMA,
            pltpu.SemaphoreType.DMA,
            pltpu.SemaphoreType.DMA,
        ],
    )
    def pq_decode(ids_hbm, indexes_hbm, codes_hbm, out_hbm,
                  ids_v, idx_a, idx_b, codes_v, out_a, out_b,
                  sg_a, sg_b, so_a, so_b):
        nc = 2
        wid = lax.axis_index("s") * nc + lax.axis_index("c")
        base = wid * per_w

        pltpu.sync_copy(codes_hbm, codes_v)
        pltpu.sync_copy(ids_hbm.at[pl.ds(base, per_w)], ids_v)

        q_iota = lax.iota(jnp.int32, 16)
        q_base = q_iota * (centroids * chunk)
        o_iota = q_iota * chunk

        idx_bufs = (idx_a, idx_b)
        out_bufs = (out_a, out_b)
        sg = (sg_a, sg_b)
        so = (so_a, so_b)

        def gather_start(i, buf, sem):
            pltpu.async_copy(
                indexes_hbm.at[ids_v.at[pl.ds(i * ch, ch)]], buf, sem)

        def gather_wait(i, buf, sem):
            pltpu.make_async_copy(
                indexes_hbm.at[ids_v.at[pl.ds(i * ch, ch)]], buf, sem).wait()

        def write_start(i, buf, sem):
            pltpu.async_copy(
                buf, out_hbm.at[pl.ds((base + i * ch) * dim, ch * dim)], sem)

        def write_wait(i, buf, sem):
            pltpu.make_async_copy(
                buf, out_hbm.at[pl.ds((base + i * ch) * dim, ch * dim)],
                sem).wait()

        gather_start(0, idx_a, sg_a)

        def chunk_body(j, carry):
            for b in range(2):
                i = j * 2 + b
                idx_v = idx_bufs[b]
                out_v = out_bufs[b]

                @pl.when(i + 1 < n_chunks)
                def _():
                    gather_start(i + 1, idx_bufs[1 - b], sg[1 - b])

                gather_wait(i, idx_v, sg[b])

                @pl.when(i >= 2)
                def _():
                    write_wait(i - 2, out_v, so[b])

                def id_body(v, c2):
                    iv = idx_v[v, :]
                    addr0 = q_base + iv * chunk
                    obase = o_iota + v * dim
                    for c in range(chunk):
                        vals = plsc.load_gather(codes_v, [addr0 + c])
                        plsc.store_scatter(out_v, [obase + c], vals)
                    return c2

                lax.fori_loop(0, ch, id_body, 0, unroll=2)
                write_start(i, out_v, so[b])
            return carry

        lax.fori_loop(0, n_chunks // 2, chunk_body, 0)
        write_wait(n_chunks - 2, out_a, so_a)
        write_wait(n_chunks - 1, out_b, so_b)

    return pq_decode


def kernel(input, indexes, codes):
    shape = input.shape
    qdim, centroids, chunk = codes.shape
    dim = qdim * chunk
    flat = input.reshape(-1)
    V = flat.shape[0]
    per_w = V // 32
    ch = 128
    fn = _build_kernel(V, qdim, centroids, chunk, per_w, ch)
    out = fn(flat, indexes, codes.reshape(-1))
    return out.reshape(shape + (dim,))


# speedup vs baseline: 454.7767x; 1.3498x over previous
import functools

import jax
import jax.numpy as jnp
from jax import lax
from jax.experimental import pallas as pl
from jax.experimental.pallas import tpu as pltpu
from jax.experimental.pallas import tpu_sc as plsc


def _build_kernel(B, F, qdim, centroids, chunk, rows_w, rows_ch):
    dim = qdim * chunk
    ch = rows_ch * F
    per_w = rows_w * F
    n_chunks = rows_w // rows_ch
    assert n_chunks % 2 == 0
    recip_shift = 16
    recip = (1 << recip_shift) // F + 1
    assert all((v * recip) >> recip_shift == v // F for v in range(ch))
    mesh = plsc.VectorSubcoreMesh(core_axis_name="c", subcore_axis_name="s")

    @functools.partial(
        pl.kernel,
        mesh=mesh,
        compiler_params=pltpu.CompilerParams(
            needs_layout_passes=False, use_tc_tiling_on_sc=False),
        out_type=jax.ShapeDtypeStruct((B, F, dim), jnp.float32),
        scratch_types=[
            pltpu.VMEM((per_w,), jnp.int32),
            pltpu.VMEM((ch, qdim), jnp.int32),
            pltpu.VMEM((ch, qdim), jnp.int32),
            pltpu.VMEM((qdim * centroids * chunk,), jnp.float32),
            pltpu.VMEM((rows_ch, F, dim), jnp.float32),
            pltpu.VMEM((rows_ch, F, dim), jnp.float32),
            pltpu.SemaphoreType.DMA,
            pltpu.SemaphoreType.DMA,
            pltpu.SemaphoreType.DMA,
            pltpu.SemaphoreType.DMA,
        ],
    )
    def pq_decode(ids_hbm, indexes_hbm, codes_hbm, out_hbm,
                  ids_v, idx_a, idx_b, codes_v, out_a, out_b,
                  sg_a, sg_b, so_a, so_b):
        nc = 2
        wid = lax.axis_index("s") * nc + lax.axis_index("c")
        base = wid * per_w
        row_base = wid * rows_w

        pltpu.sync_copy(codes_hbm, codes_v)
        pltpu.sync_copy(ids_hbm.at[pl.ds(base, per_w)], ids_v)

        q_iota = lax.iota(jnp.int32, 16)
        q_bases = [q_iota * (centroids * chunk) + c for c in range(chunk)]
        o_cols = [q_iota * chunk + c for c in range(chunk)]

        idx_bufs = (idx_a, idx_b)
        out_bufs = (out_a, out_b)
        sg = (sg_a, sg_b)
        so = (so_a, so_b)

        def gather_start(i, buf, sem):
            pltpu.async_copy(
                indexes_hbm.at[ids_v.at[pl.ds(i * ch, ch)]], buf, sem)

        def gather_wait(i, buf, sem):
            pltpu.make_async_copy(
                indexes_hbm.at[ids_v.at[pl.ds(i * ch, ch)]], buf, sem).wait()

        def write_start(i, buf, sem):
            pltpu.async_copy(
                buf, out_hbm.at[pl.ds(row_base + i * rows_ch, rows_ch)], sem)

        def write_wait(i, buf, sem):
            pltpu.make_async_copy(
                buf, out_hbm.at[pl.ds(row_base + i * rows_ch, rows_ch)],
                sem).wait()

        gather_start(0, idx_a, sg_a)

        def chunk_body(j, carry):
            for b in range(2):
                i = j * 2 + b
                idx_v = idx_bufs[b]
                out_v = out_bufs[b]

                @pl.when(i + 1 < n_chunks)
                def _():
                    gather_start(i + 1, idx_bufs[1 - b], sg[1 - b])

                gather_wait(i, idx_v, sg[b])

                @pl.when(i >= 2)
                def _():
                    write_wait(i - 2, out_v, so[b])

                @plsc.parallel_loop(0, ch, unroll=4)
                def _(v):
                    iv = idx_v[v, :]
                    base4 = iv * chunk
                    r = (v * recip) >> recip_shift
                    f = v - r * F
                    rvec = jnp.full((16,), r, dtype=jnp.int32)
                    fvec = jnp.full((16,), f, dtype=jnp.int32)
                    for c in range(chunk):
                        vals = plsc.load_gather(codes_v, [base4 + q_bases[c]])
                        plsc.store_scatter(
                            out_v, [rvec, fvec, o_cols[c]], vals)

                write_start(i, out_v, so[b])
            return carry

        lax.fori_loop(0, n_chunks // 2, chunk_body, 0)
        write_wait(n_chunks - 2, out_a, so_a)
        write_wait(n_chunks - 1, out_b, so_b)

    return pq_decode


def kernel(input, indexes, codes):
    shape = input.shape
    qdim, centroids, chunk = codes.shape
    B, F = shape
    flat = input.reshape(-1)
    rows_w = B // 32
    rows_ch = 4
    fn = _build_kernel(B, F, qdim, centroids, chunk, rows_w, rows_ch)
    return fn(flat, indexes, codes.reshape(-1))


# speedup vs baseline: 466.9920x; 1.0269x over previous
import functools

import jax
import jax.numpy as jnp
from jax import lax
from jax.experimental import pallas as pl
from jax.experimental.pallas import tpu as pltpu
from jax.experimental.pallas import tpu_sc as plsc


def _build_kernel(B, F, qdim, centroids, chunk, rows_w, rows_ch):
    dim = qdim * chunk
    ch = rows_ch * F
    per_w = rows_w * F
    n_chunks = rows_w // rows_ch
    nbuf = 4
    assert n_chunks % nbuf == 0
    recip_shift = 16
    recip = (1 << recip_shift) // F + 1
    assert all((v * recip) >> recip_shift == v // F for v in range(ch))
    mesh = plsc.VectorSubcoreMesh(core_axis_name="c", subcore_axis_name="s")

    @functools.partial(
        pl.kernel,
        mesh=mesh,
        compiler_params=pltpu.CompilerParams(
            needs_layout_passes=False, use_tc_tiling_on_sc=False),
        out_type=jax.ShapeDtypeStruct((B, F, dim), jnp.float32),
        scratch_types=(
            [pltpu.VMEM((per_w,), jnp.int32)]
            + [pltpu.VMEM((ch, qdim), jnp.int32) for _ in range(nbuf)]
            + [pltpu.VMEM((qdim * centroids * chunk,), jnp.float32)]
            + [pltpu.VMEM((rows_ch, F, dim), jnp.float32) for _ in range(nbuf)]
            + [pltpu.SemaphoreType.DMA for _ in range(2 * nbuf)]
        ),
    )
    def pq_decode(ids_hbm, indexes_hbm, codes_hbm, out_hbm, ids_v, *rest):
        idx_bufs = rest[0:nbuf]
        codes_v = rest[nbuf]
        out_bufs = rest[nbuf + 1:2 * nbuf + 1]
        sg = rest[2 * nbuf + 1:3 * nbuf + 1]
        so = rest[3 * nbuf + 1:4 * nbuf + 1]

        nc = 2
        wid = lax.axis_index("s") * nc + lax.axis_index("c")
        base = wid * per_w
        row_base = wid * rows_w

        pltpu.sync_copy(codes_hbm, codes_v)
        pltpu.sync_copy(ids_hbm.at[pl.ds(base, per_w)], ids_v)

        q_iota = lax.iota(jnp.int32, 16)
        q_bases = [q_iota * (centroids * chunk) + c for c in range(chunk)]
        o_cols = [q_iota * chunk + c for c in range(chunk)]

        def gather_start(i, buf, sem):
            pltpu.async_copy(
                indexes_hbm.at[ids_v.at[pl.ds(i * ch, ch)]], buf, sem)

        def gather_wait(i, buf, sem):
            pltpu.make_async_copy(
                indexes_hbm.at[ids_v.at[pl.ds(i * ch, ch)]], buf, sem).wait()

        def write_start(i, buf, sem):
            pltpu.async_copy(
                buf, out_hbm.at[pl.ds(row_base + i * rows_ch, rows_ch)], sem)

        def write_wait(i, buf, sem):
            pltpu.make_async_copy(
                buf, out_hbm.at[pl.ds(row_base + i * rows_ch, rows_ch)],
                sem).wait()

        for p in range(nbuf - 1):
            gather_start(p, idx_bufs[p], sg[p])

        def chunk_body(j, carry):
            for b in range(nbuf):
                i = j * nbuf + b
                idx_v = idx_bufs[b]
                out_v = out_bufs[b]

                @pl.when(i + nbuf - 1 < n_chunks)
                def _():
                    gather_start(i + nbuf - 1, idx_bufs[(b + nbuf - 1) % nbuf],
                                 sg[(b + nbuf - 1) % nbuf])

                gather_wait(i, idx_v, sg[b])

                @pl.when(i >= nbuf)
                def _():
                    write_wait(i - nbuf, out_v, so[b])

                @plsc.parallel_loop(0, ch, unroll=4)
                def _(v):
                    iv = idx_v[v, :]
                    base4 = iv * chunk
                    r = (v * recip) >> recip_shift
                    f = v - r * F
                    rvec = jnp.full((16,), r, dtype=jnp.int32)
                    fvec = jnp.full((16,), f, dtype=jnp.int32)
                    for c in range(chunk):
                        vals = plsc.load_gather(codes_v, [base4 + q_bases[c]])
                        plsc.store_scatter(
                            out_v, [rvec, fvec, o_cols[c]], vals)

                write_start(i, out_v, so[b])
            return carry

        lax.fori_loop(0, n_chunks // nbuf, chunk_body, 0)
        for p in range(nbuf):
            write_wait(n_chunks - nbuf + p, out_bufs[p], so[p])

    return pq_decode


def kernel(input, indexes, codes):
    shape = input.shape
    qdim, centroids, chunk = codes.shape
    B, F = shape
    flat = input.reshape(-1)
    rows_w = B // 32
    rows_ch = 4
    fn = _build_kernel(B, F, qdim, centroids, chunk, rows_w, rows_ch)
    return fn(flat, indexes, codes.reshape(-1))
